# Initial kernel scaffold; baseline (speedup 1.0000x reference)
import jax
import jax.numpy as jnp
from jax.experimental import pallas as pl


def kernel(x, edge_index, W1_l, b1, W1_r, W2_l, b2, W2_r):
    raise NotImplementedError("write your pallas kernel here")


# speedup vs baseline: 13.6443x; 13.6443x over previous
import functools

import jax
import jax.numpy as jnp
from jax import lax
from jax.experimental import pallas as pl
from jax.experimental.pallas import tpu as pltpu
from jax.experimental.pallas import tpu_sc as plsc

NC = 2
NS = 16
NW = NC * NS


def _proj1(x, W1_l, W1_r, b1):
    n, _ = x.shape
    d_h = W1_l.shape[1]

    def body(x_ref, wl_ref, wr_ref, b_ref, y_ref, z_ref):
        xx = x_ref[...]
        y_ref[...] = jnp.dot(xx, wl_ref[...], preferred_element_type=jnp.float32)
        z_ref[...] = (
            jnp.dot(xx, wr_ref[...], preferred_element_type=jnp.float32)
            + b_ref[...]
        )

    return pl.pallas_call(
        body,
        out_shape=[
            jax.ShapeDtypeStruct((n, d_h), jnp.float32),
            jax.ShapeDtypeStruct((n, d_h), jnp.float32),
        ],
    )(x, W1_l, W1_r, b1.reshape(1, d_h))


def _combine1(s1p, degp, z1):
    n, d_h = z1.shape

    def body(s_ref, d_ref, z_ref, h_ref, r_ref):
        s = s_ref[0] + s_ref[1]
        r = 1.0 / jnp.maximum(d_ref[0] + d_ref[1], 1.0)
        h_ref[...] = jnp.maximum(s * r + z_ref[...], 0.0)
        r_ref[...] = r

    return pl.pallas_call(
        body,
        out_shape=[
            jax.ShapeDtypeStruct((n, d_h), jnp.float32),
            jax.ShapeDtypeStruct((n, d_h), jnp.float32),
        ],
    )(s1p, degp, z1)


def _combine2(s2p, rdeg, h, W2_l, W2_r, b2):
    n, d_h = h.shape
    d_out = W2_l.shape[1]

    def body(s_ref, r_ref, h_ref, wl_ref, wr_ref, b_ref, o_ref):
        m = (s_ref[0] + s_ref[1]) * r_ref[...]
        o_ref[...] = (
            jnp.dot(m, wl_ref[...], preferred_element_type=jnp.float32)
            + jnp.dot(h_ref[...], wr_ref[...], preferred_element_type=jnp.float32)
            + b_ref[...]
        )

    return pl.pallas_call(
        body,
        out_shape=jax.ShapeDtypeStruct((n, d_out), jnp.float32),
    )(s2p, rdeg, h, W2_l, W2_r, b2.reshape(1, d_out))


@functools.partial(jax.jit, static_argnames=("with_deg",))
def _sc_segment_sum(y, srcr, dstr, *, with_deg):
    n, d = y.shape
    _, num_chunks, batch = srcr.shape
    rows_per_tile = n // NS

    mesh = plsc.VectorSubcoreMesh(
        core_axis_name="c", subcore_axis_name="s", num_cores=NC, num_subcores=NS
    )
    out_type = [jax.ShapeDtypeStruct((NC, n, d), jnp.float32)]
    scratch = [
        pltpu.VMEM((num_chunks, batch), jnp.int32),
        pltpu.VMEM((num_chunks, batch), jnp.int32),
        pltpu.VMEM((batch, d), jnp.float32),
        pltpu.VMEM((rows_per_tile, d), jnp.float32),
        pltpu.VMEM_SHARED((n, d), jnp.float32),
        pltpu.SemaphoreType.DMA,
    ]
    if with_deg:
        out_type.append(jax.ShapeDtypeStruct((NC, n, d), jnp.float32))
        scratch.append(pltpu.VMEM((batch, d), jnp.float32))
        scratch.append(pltpu.VMEM_SHARED((n, d), jnp.float32))

    @functools.partial(
        pl.kernel, out_type=out_type, mesh=mesh, scratch_types=scratch,
        compiler_params=pltpu.CompilerParams(use_tc_tiling_on_sc=False),
    )
    def k(y_hbm, src_hbm, dst_hbm, *refs):
        if with_deg:
            (s_out, deg_out, idxs_v, idxd_v, rows_v, zbuf_v, acc_s, sem,
             ones_v, acc_d) = refs
        else:
            s_out, idxs_v, idxd_v, rows_v, zbuf_v, acc_s, sem = refs

        cid = lax.axis_index("c")
        sid = lax.axis_index("s")
        wid = cid * NS + sid
        base = sid * rows_per_tile

        zv = jnp.zeros((d,), jnp.float32)

        @pl.loop(0, rows_per_tile)
        def _(i):
            zbuf_v[i, :] = zv

        if with_deg:
            ov = jnp.ones((d,), jnp.float32)

            @pl.loop(0, batch)
            def _(i):
                ones_v[i, :] = ov

        pltpu.sync_copy(zbuf_v, acc_s.at[pl.ds(base, rows_per_tile)])
        if with_deg:
            pltpu.sync_copy(zbuf_v, acc_d.at[pl.ds(base, rows_per_tile)])
        pltpu.sync_copy(src_hbm.at[wid], idxs_v)
        pltpu.sync_copy(dst_hbm.at[wid], idxd_v)
        plsc.subcore_barrier()

        @pl.loop(0, num_chunks)
        def _(kk):
            pltpu.async_copy(y_hbm.at[idxs_v.at[kk]], rows_v, sem).wait()
            pltpu.sync_copy(rows_v, acc_s.at[idxd_v.at[kk]], add=True)
            if with_deg:
                pltpu.sync_copy(ones_v, acc_d.at[idxd_v.at[kk]], add=True)

        plsc.subcore_barrier()
        sl = pl.ds(base, rows_per_tile)
        pltpu.sync_copy(acc_s.at[sl], s_out.at[cid, sl])
        if with_deg:
            pltpu.sync_copy(acc_d.at[sl], deg_out.at[cid, sl])

    return k(y, srcr, dstr)


def kernel(x, edge_index, W1_l, b1, W1_r, W2_l, b2, W2_r):
    n = x.shape[0]
    e = edge_index.shape[1]
    per_tile = e // NW
    assert per_tile * NW == e
    batch = 128
    while per_tile % batch:
        batch -= 1
    num_chunks = per_tile // batch
    n_pad = -(-n // (NS * 8)) * (NS * 8)

    srcr = edge_index[0].reshape(NW, num_chunks, batch)
    dstr = edge_index[1].reshape(NW, num_chunks, batch)

    y1, z1 = _proj1(x, W1_l, W1_r, b1)
    y1 = jnp.pad(y1, ((0, n_pad - n), (0, 0)))
    z1 = jnp.pad(z1, ((0, n_pad - n), (0, 0)))
    s1p, degp = _sc_segment_sum(y1, srcr, dstr, with_deg=True)
    h, rdeg = _combine1(s1p, degp, z1)
    (s2p,) = _sc_segment_sum(h, srcr, dstr, with_deg=False)
    return _combine2(s2p, rdeg, h, W2_l, W2_r, b2)[:n]


# speedup vs baseline: 18.5840x; 1.3620x over previous
import functools

import jax
import jax.numpy as jnp
from jax import lax
from jax.experimental import pallas as pl
from jax.experimental.pallas import tpu as pltpu
from jax.experimental.pallas import tpu_sc as plsc

NC = 2
NS = 16
NW = NC * NS


def _proj1(x, W1_l, W1_r, b1):
    n, _ = x.shape
    d_h = W1_l.shape[1]

    def body(x_ref, wl_ref, wr_ref, b_ref, y_ref, z_ref):
        xx = x_ref[...]
        y_ref[...] = jnp.dot(xx, wl_ref[...], preferred_element_type=jnp.float32)
        z_ref[...] = (
            jnp.dot(xx, wr_ref[...], preferred_element_type=jnp.float32)
            + b_ref[...]
        )

    return pl.pallas_call(
        body,
        out_shape=[
            jax.ShapeDtypeStruct((n, d_h), jnp.float32),
            jax.ShapeDtypeStruct((n, d_h), jnp.float32),
        ],
    )(x, W1_l, W1_r, b1.reshape(1, d_h))


def _combine1(s1p, degp, z1):
    n, d_h = z1.shape

    def body(s_ref, d_ref, z_ref, h_ref, r_ref):
        s = s_ref[0] + s_ref[1]
        r = 1.0 / jnp.maximum(d_ref[0] + d_ref[1], 1.0)
        h_ref[...] = jnp.maximum(s * r + z_ref[...], 0.0)
        r_ref[...] = r

    return pl.pallas_call(
        body,
        out_shape=[
            jax.ShapeDtypeStruct((n, d_h), jnp.float32),
            jax.ShapeDtypeStruct((n, d_h), jnp.float32),
        ],
    )(s1p, degp, z1)


def _combine2(s2p, rdeg, h, W2_l, W2_r, b2):
    n, d_h = h.shape
    d_out = W2_l.shape[1]

    def body(s_ref, r_ref, h_ref, wl_ref, wr_ref, b_ref, o_ref):
        m = (s_ref[0] + s_ref[1]) * r_ref[...]
        o_ref[...] = (
            jnp.dot(m, wl_ref[...], preferred_element_type=jnp.float32)
            + jnp.dot(h_ref[...], wr_ref[...], preferred_element_type=jnp.float32)
            + b_ref[...]
        )

    return pl.pallas_call(
        body,
        out_shape=jax.ShapeDtypeStruct((n, d_out), jnp.float32),
    )(s2p, rdeg, h, W2_l, W2_r, b2.reshape(1, d_out))


@functools.partial(jax.jit, static_argnames=("with_deg",))
def _sc_segment_sum(y, srcr, dstr, *, with_deg):
    n, d = y.shape
    _, num_chunks, batch = srcr.shape
    rows_per_tile = n // NS

    mesh = plsc.VectorSubcoreMesh(
        core_axis_name="c", subcore_axis_name="s", num_cores=NC, num_subcores=NS
    )
    assert num_chunks % 2 == 0
    out_type = [jax.ShapeDtypeStruct((NC, n, d), jnp.float32)]
    scratch = [
        pltpu.VMEM((num_chunks, batch), jnp.int32),
        pltpu.VMEM((num_chunks, batch), jnp.int32),
        pltpu.VMEM((batch, d), jnp.float32),
        pltpu.VMEM((batch, d), jnp.float32),
        pltpu.VMEM((rows_per_tile, d), jnp.float32),
        pltpu.VMEM_SHARED((n, d), jnp.float32),
        pltpu.SemaphoreType.DMA,
        pltpu.SemaphoreType.DMA,
    ]
    if with_deg:
        out_type.append(jax.ShapeDtypeStruct((NC, n, d), jnp.float32))
        scratch.append(pltpu.VMEM((batch, d), jnp.float32))
        scratch.append(pltpu.VMEM_SHARED((n, d), jnp.float32))

    @functools.partial(
        pl.kernel, out_type=out_type, mesh=mesh, scratch_types=scratch,
        compiler_params=pltpu.CompilerParams(use_tc_tiling_on_sc=False),
    )
    def k(y_hbm, src_hbm, dst_hbm, *refs):
        if with_deg:
            (s_out, deg_out, idxs_v, idxd_v, rows_a, rows_b, zbuf_v, acc_s,
             sem_a, sem_b, ones_v, acc_d) = refs
        else:
            (s_out, idxs_v, idxd_v, rows_a, rows_b, zbuf_v, acc_s,
             sem_a, sem_b) = refs

        cid = lax.axis_index("c")
        sid = lax.axis_index("s")
        wid = cid * NS + sid
        base = sid * rows_per_tile

        zv = jnp.zeros((d,), jnp.float32)

        @pl.loop(0, rows_per_tile)
        def _(i):
            zbuf_v[i, :] = zv

        if with_deg:
            ov = jnp.ones((d,), jnp.float32)

            @pl.loop(0, batch)
            def _(i):
                ones_v[i, :] = ov

        pltpu.sync_copy(zbuf_v, acc_s.at[pl.ds(base, rows_per_tile)])
        if with_deg:
            pltpu.sync_copy(zbuf_v, acc_d.at[pl.ds(base, rows_per_tile)])
        pltpu.sync_copy(src_hbm.at[wid], idxs_v)
        pltpu.sync_copy(dst_hbm.at[wid], idxd_v)
        plsc.subcore_barrier()

        pltpu.async_copy(y_hbm.at[idxs_v.at[0]], rows_a, sem_a)

        @pl.loop(0, num_chunks, step=2)
        def _(kk):
            pltpu.async_copy(y_hbm.at[idxs_v.at[kk + 1]], rows_b, sem_b)
            pltpu.make_async_copy(y_hbm.at[idxs_v.at[kk]], rows_a, sem_a).wait()
            pltpu.sync_copy(rows_a, acc_s.at[idxd_v.at[kk]], add=True)
            if with_deg:
                pltpu.sync_copy(ones_v, acc_d.at[idxd_v.at[kk]], add=True)

            @pl.when(kk + 2 < num_chunks)
            def _():
                pltpu.async_copy(y_hbm.at[idxs_v.at[kk + 2]], rows_a, sem_a)

            pltpu.make_async_copy(
                y_hbm.at[idxs_v.at[kk + 1]], rows_b, sem_b
            ).wait()
            pltpu.sync_copy(rows_b, acc_s.at[idxd_v.at[kk + 1]], add=True)
            if with_deg:
                pltpu.sync_copy(ones_v, acc_d.at[idxd_v.at[kk + 1]], add=True)

        plsc.subcore_barrier()
        sl = pl.ds(base, rows_per_tile)
        pltpu.sync_copy(acc_s.at[sl], s_out.at[cid, sl])
        if with_deg:
            pltpu.sync_copy(acc_d.at[sl], deg_out.at[cid, sl])

    return k(y, srcr, dstr)


def kernel(x, edge_index, W1_l, b1, W1_r, W2_l, b2, W2_r):
    n = x.shape[0]
    e = edge_index.shape[1]
    per_tile = e // NW
    assert per_tile * NW == e
    batch = 128
    while per_tile % batch:
        batch -= 1
    num_chunks = per_tile // batch
    n_pad = -(-n // (NS * 8)) * (NS * 8)

    srcr = edge_index[0].reshape(NW, num_chunks, batch)
    dstr = edge_index[1].reshape(NW, num_chunks, batch)

    y1, z1 = _proj1(x, W1_l, W1_r, b1)
    y1 = jnp.pad(y1, ((0, n_pad - n), (0, 0)))
    z1 = jnp.pad(z1, ((0, n_pad - n), (0, 0)))
    s1p, degp = _sc_segment_sum(y1, srcr, dstr, with_deg=True)
    h, rdeg = _combine1(s1p, degp, z1)
    (s2p,) = _sc_segment_sum(h, srcr, dstr, with_deg=False)
    return _combine2(s2p, rdeg, h, W2_l, W2_r, b2)[:n]
